# baseline (speedup 1.0000x reference)
import functools

import jax
import jax.numpy as jnp
import numpy as np
from jax import lax
from jax.experimental import pallas as pl
from jax.experimental.pallas import tpu as pltpu
from jax.experimental.pallas import tpu_sc as plsc

_NUM_LEVELS = 8
_LOG2_T = 18
_T = 1 << _LOG2_T
_BASE_RES = 16
_MAX_RES = 1024
_SCALE = np.exp(np.log(_MAX_RES / _BASE_RES) / (_NUM_LEVELS - 1))
_RES = [int(np.floor(_BASE_RES * _SCALE ** lv)) for lv in range(_NUM_LEVELS)]
_P1 = int(np.int32(np.uint32(2654435761)))
_P2 = int(np.int32(805459861))
_MASK = _T - 1

_NC, _NS, _L = 2, 16, 16
_NW = _NC * _NS

_C = 32
_FD = 2
_PAD = 8

_NDL = 2
_NH = _NUM_LEVELS - _NDL
_DS = tuple(_RES[lv] + 2 for lv in range(_NDL))
_DWORDS = tuple(2 * s * s * s + 16 for s in _DS)


def _pad_body(c0h, c1h, padh, c0b, c1b, rowb):
    nrows = padh.shape[0] - 7 * _T
    per_w = nrows // _NW
    kc = c0b.shape[0]
    wid = lax.axis_index("s") * _NC + lax.axis_index("c")
    iota = lax.iota(jnp.int32, _L)
    zeros = iota * 0
    ones = zeros + 1

    @pl.loop(0, per_w // kc)
    def _chunk(k):
        base = wid * per_w + k * kc
        pltpu.sync_copy(c0h.at[pl.ds(base, kc)], c0b)
        pltpu.sync_copy(c1h.at[pl.ds(base, kc)], c1b)

        @pl.loop(0, kc // _L)
        def _g(g):
            rows = g * _L + iota
            plsc.store_scatter(rowb, [rows, zeros], c0b[pl.ds(g * _L, _L)])
            plsc.store_scatter(rowb, [rows, ones], c1b[pl.ds(g * _L, _L)])

        pltpu.sync_copy(rowb, padh.at[pl.ds(base, kc)])

    lb = 2 * _T + wid * kc
    pltpu.sync_copy(c0h.at[pl.ds(lb, kc)], c0b)
    pltpu.sync_copy(c1h.at[pl.ds(lb, kc)], c1b)

    @pl.loop(0, kc // _L)
    def _g2(g):
        rows = g * _L + iota
        plsc.store_scatter(rowb, [rows, zeros], c0b[pl.ds(g * _L, _L)])
        plsc.store_scatter(rowb, [rows, ones], c1b[pl.ds(g * _L, _L)])

    for rep in range(7):
        pltpu.sync_copy(
            rowb, padh.at[pl.ds((8 + rep) * _T + wid * kc, kc)]
        )


def _pad_table(c0, c1):
    nrows = c0.shape[0]
    kc = 8192
    mesh = plsc.VectorSubcoreMesh(
        core_axis_name="c", subcore_axis_name="s", num_cores=_NC, num_subcores=_NS
    )
    f = pl.kernel(
        _pad_body,
        out_type=jax.ShapeDtypeStruct((nrows + 7 * _T, _PAD), jnp.float32),
        mesh=mesh,
        scratch_types=[
            pltpu.VMEM((kc,), jnp.float32),
            pltpu.VMEM((kc,), jnp.float32),
            pltpu.VMEM((kc, _PAD), jnp.float32),
        ],
        compiler_params=pltpu.CompilerParams(
            needs_layout_passes=False, use_tc_tiling_on_sc=False
        ),
    )
    return f(c0, c1)


def _encode_body(
    x0h, x1h, x2h, tabh, ench,
    xb, wb, cidxb, idxb, featb, outb, d0b, d1b, sem0, sem1, xsem,
):
    n = x0h.shape[0]
    npts = n // _NW
    nchunks = npts // _C
    wid = lax.axis_index("s") * _NC + lax.axis_index("c")

    iota = lax.iota(jnp.int32, _L)
    dup = iota >> 1
    par = iota & 1
    sems = (sem0, sem1)

    rep = (iota + wid) & 7
    lv2base = jnp.where(rep == 0, jnp.int32(2), rep + 7) << _LOG2_T

    def fire_x(k, p):
        base = wid * npts + k * _C
        o = p * 3 * _C
        c0 = pltpu.make_async_copy(
            x0h.at[pl.ds(base, _C)], xb.at[pl.ds(o, _C)], xsem)
        c1 = pltpu.make_async_copy(
            x1h.at[pl.ds(base, _C)], xb.at[pl.ds(o + _C, _C)], xsem)
        c2 = pltpu.make_async_copy(
            x2h.at[pl.ds(base, _C)], xb.at[pl.ds(o + 2 * _C, _C)], xsem)
        c0.start(); c1.start(); c2.start()

    def wait_x(k, p):
        base = wid * npts + k * _C
        o = p * 3 * _C
        pltpu.make_async_copy(
            x0h.at[pl.ds(base, _C)], xb.at[pl.ds(o, _C)], xsem).wait()
        pltpu.make_async_copy(
            x1h.at[pl.ds(base, _C)], xb.at[pl.ds(o + _C, _C)], xsem).wait()
        pltpu.make_async_copy(
            x2h.at[pl.ds(base, _C)], xb.at[pl.ds(o + 2 * _C, _C)], xsem).wait()

    def gather_copy(p):
        return pltpu.make_async_copy(
            tabh.at[idxb.at[pl.ds(p * 8 * _NH * _C, 8 * _NH * _C)]],
            featb.at[pl.ds(p * 8 * _NH * _C, 8 * _NH * _C)],
            sems[p],
        )

    def build_dense(dref, s, lv):
        s2 = s * s
        ng = (2 * s2 + _L - 1) // _L
        lvoff = lv << _LOG2_T

        @pl.loop(0, s)
        def _z(z):
            @pl.loop(0, s)
            def _y(y):
                hs = (y * _P1) ^ (z * _P2)
                pos0 = y * s
                h1 = ((iota ^ hs) & _MASK) | lvoff
                plsc.store_scatter(idxb, [pos0 + iota], h1)
                h2 = (((iota + _L) ^ hs) & _MASK) | lvoff
                plsc.store_scatter(
                    idxb, [pos0 + _L + iota], h2, mask=iota < (s - _L)
                )

            cp = pltpu.make_async_copy(
                tabh.at[idxb.at[pl.ds(0, s2)]], featb.at[pl.ds(0, s2)], sem0
            )
            cp.start()
            cp.wait()

            @pl.loop(0, ng)
            def _g(g):
                v = plsc.load_gather(featb, [g * 8 + dup, par])
                dref[pl.ds(z * 2 * s2 + g * _L, _L)] = v

    def pass1(p):
        xo = p * 3 * _C
        io = p * 8 * _NH * _C
        wo = p * 24 * _C
        co = p * _NDL * _C

        @pl.loop(0, _C // _L)
        def _p1(g):
            off = g * _L
            xr0 = (xb[pl.ds(xo + off, _L)] + 1.0) * 0.5
            xr1 = (xb[pl.ds(xo + _C + off, _L)] + 1.0) * 0.5
            xr2 = (xb[pl.ds(xo + 2 * _C + off, _L)] + 1.0) * 0.5
            for lv in range(_NUM_LEVELS):
                r = float(_RES[lv])
                iv = []
                for d, xr in enumerate((xr0, xr1, xr2)):
                    pos = xr * r
                    i_d = pos.astype(jnp.int32)
                    t = pos - i_d.astype(jnp.float32)
                    w = t * t * (3.0 - 2.0 * t)
                    wb[pl.ds(wo + lv * 3 * _C + d * _C + off, _L)] = w
                    iv.append(i_d)
                if lv < _NDL:
                    s = _DS[lv]
                    base2 = 2 * (iv[0] + s * iv[1] + (s * s) * iv[2])
                    cidxb[pl.ds(co + lv * _C + off, _L)] = base2
                else:
                    b0 = iv[1] * _P1
                    b1 = b0 + _P1
                    c0 = iv[2] * _P2
                    c1 = c0 + _P2
                    ab = (iv[0] ^ b0, (iv[0] + 1) ^ b0, iv[0] ^ b1,
                          (iv[0] + 1) ^ b1)
                    lvoff = lv2base if lv == 2 else (lv << _LOG2_T)
                    j = lv - _NDL
                    corner = 0
                    for cz in (c0, c1):
                        for abx in ab:
                            h = ((abx ^ cz) & _MASK) | lvoff
                            idxb[pl.ds(io + (j * 8 + corner) * _C + off, _L)] = h
                            corner += 1

    dense_refs = (d0b, d1b)

    def pass2(k, p):
        wo = p * 24 * _C
        fo = p * 8 * _NH * _C
        co = p * _NDL * _C
        cbase = (k % (128 // _C)) * _C

        @pl.loop(0, _C // 8)
        def _p2(j):
            p0 = j * 8
            rowv = p0 + dup
            wv = wo + p0 + dup
            for lv in range(_NUM_LEVELS):
                w = [
                    plsc.load_gather(wb, [wv + (lv * 3 * _C + d * _C)])
                    for d in range(3)
                ]
                if lv < _NDL:
                    s = _DS[lv]
                    bd = plsc.load_gather(cidxb, [co + lv * _C + p0 + dup])
                    bb = bd + par
                    f = [
                        plsc.load_gather(
                            dense_refs[lv],
                            [bb + 2 * (dx + s * dy + s * s * dz)],
                        )
                        for dz in (0, 1)
                        for dy in (0, 1)
                        for dx in (0, 1)
                    ]
                else:
                    jh = lv - _NDL
                    f = [
                        plsc.load_gather(
                            featb, [(fo + (jh * 8 + c) * _C + p0) + dup, par]
                        )
                        for c in range(8)
                    ]
                g00 = f[0] + w[0] * (f[1] - f[0])
                g10 = f[2] + w[0] * (f[3] - f[2])
                g01 = f[4] + w[0] * (f[5] - f[4])
                g11 = f[6] + w[0] * (f[7] - f[6])
                h0 = g00 + w[1] * (g10 - g00)
                h1 = g01 + w[1] * (g11 - g01)
                acc = h0 + w[2] * (h1 - h0)
                plsc.store_scatter(outb, [(2 * lv) + par, cbase + rowv], acc)

    def flush_out(k):
        base = wid * npts + (k - (128 // _C - 1)) * _C
        j0 = base // 128
        pltpu.sync_copy(outb.at[pl.ds(0, 8), :], ench.at[0, j0])
        pltpu.sync_copy(outb.at[pl.ds(8, 8), :], ench.at[1, j0])

    build_dense(d0b, _DS[0], 0)
    build_dense(d1b, _DS[1], 1)

    fire_x(0, 0)
    wait_x(0, 0)
    pass1(0)
    gather_copy(0).start()
    fire_x(1, 1)

    @pl.loop(0, nchunks // 2)
    def _chunk2(kk):
        ka = 2 * kk
        kb = 2 * kk + 1
        wait_x(kb, 1)
        pass1(1)
        gather_copy(1).start()

        @pl.when(kk < nchunks // 2 - 1)
        def _():
            fire_x(ka + 2, 0)

        gather_copy(0).wait()
        pass2(ka, 0)

        @pl.when(kk < nchunks // 2 - 1)
        def _():
            wait_x(ka + 2, 0)
            pass1(0)
            gather_copy(0).start()
            fire_x(kb + 2, 1)

        gather_copy(1).wait()
        pass2(kb, 1)

        @pl.when(kb % 4 == 3)
        def _():
            flush_out(kb)


def _encode(x0, x1, x2, table2):
    n = x0.shape[0]
    mesh = plsc.VectorSubcoreMesh(
        core_axis_name="c", subcore_axis_name="s", num_cores=_NC, num_subcores=_NS
    )
    f = pl.kernel(
        _encode_body,
        out_type=jax.ShapeDtypeStruct((2, n // 128, 8, 128), jnp.float32),
        mesh=mesh,
        scratch_types=[
            pltpu.VMEM((2 * 3 * _C,), jnp.float32),
            pltpu.VMEM((2 * 24 * _C,), jnp.float32),
            pltpu.VMEM((2 * _NDL * _C,), jnp.int32),
            pltpu.VMEM((2 * 8 * _NH * _C,), jnp.int32),
            pltpu.VMEM((2 * 8 * _NH * _C, _PAD), jnp.float32),
            pltpu.VMEM((2 * _NUM_LEVELS, 128), jnp.float32),
            pltpu.VMEM((_DWORDS[0],), jnp.float32),
            pltpu.VMEM((_DWORDS[1],), jnp.float32),
            pltpu.SemaphoreType.DMA,
            pltpu.SemaphoreType.DMA,
            pltpu.SemaphoreType.DMA,
        ],
        compiler_params=pltpu.CompilerParams(
            needs_layout_passes=False, use_tc_tiling_on_sc=False
        ),
    )
    return f(x0, x1, x2, table2)


def _mlp_body(xt_ref, enc_ref, w1xt_ref, w1gt_ref, b1_ref, w2t_ref, b2_ref, out_ref):
    nblk = enc_ref.shape[1]
    enc2d = jnp.concatenate(
        [
            jnp.concatenate([enc_ref[0, j], enc_ref[1, j]], axis=0)
            for j in range(nblk)
        ],
        axis=1,
    )
    h = (
        jnp.dot(w1xt_ref[...], xt_ref[...], preferred_element_type=jnp.float32)
        + jnp.dot(w1gt_ref[...], enc2d, preferred_element_type=jnp.float32)
        + b1_ref[...]
    )
    h = jnp.maximum(h, 0.0)
    o = jnp.dot(w2t_ref[...], h, preferred_element_type=jnp.float32) + b2_ref[...]
    out_ref[...] = jnp.maximum(o, 0.0) + jnp.log1p(jnp.exp(-jnp.abs(o)))


def _mlp(xt, enc, W1, b1, W2, b2):
    n = xt.shape[1]
    bn = 4096
    d_h = W1.shape[1]
    w1t = W1.T
    out = pl.pallas_call(
        _mlp_body,
        grid=(n // bn,),
        in_specs=[
            pl.BlockSpec((3, bn), lambda i: (0, i)),
            pl.BlockSpec((2, bn // 128, 8, 128), lambda i: (0, i, 0, 0)),
            pl.BlockSpec((d_h, 3), lambda i: (0, 0)),
            pl.BlockSpec((d_h, 2 * _NUM_LEVELS), lambda i: (0, 0)),
            pl.BlockSpec((d_h, 1), lambda i: (0, 0)),
            pl.BlockSpec((1, d_h), lambda i: (0, 0)),
            pl.BlockSpec((1, 1), lambda i: (0, 0)),
        ],
        out_specs=pl.BlockSpec((1, bn), lambda i: (0, i)),
        out_shape=jax.ShapeDtypeStruct((1, n), jnp.float32),
    )(xt, enc, w1t[:, :3], w1t[:, 3:], b1.reshape(d_h, 1), W2.T, b2.reshape(1, 1))
    return out.reshape(n, 1)


def kernel(x, table, W1, b1, W2, b2):
    xt = x.T
    tpair = table.reshape(_NUM_LEVELS * _T, _FD).T
    tablep = _pad_table(tpair[0], tpair[1])
    enc = _encode(xt[0], xt[1], xt[2], tablep)
    return _mlp(xt, enc, W1, b1, W2, b2)


# speedup vs baseline: 1.0430x; 1.0430x over previous
import functools

import jax
import jax.numpy as jnp
import numpy as np
from jax import lax
from jax.experimental import pallas as pl
from jax.experimental.pallas import tpu as pltpu
from jax.experimental.pallas import tpu_sc as plsc

_NUM_LEVELS = 8
_LOG2_T = 18
_T = 1 << _LOG2_T
_BASE_RES = 16
_MAX_RES = 1024
_SCALE = np.exp(np.log(_MAX_RES / _BASE_RES) / (_NUM_LEVELS - 1))
_RES = [int(np.floor(_BASE_RES * _SCALE ** lv)) for lv in range(_NUM_LEVELS)]
_P1 = int(np.int32(np.uint32(2654435761)))
_P2 = int(np.int32(805459861))
_MASK = _T - 1

_NC, _NS, _L = 2, 16, 16
_NW = _NC * _NS

_C = 32
_FD = 2
_PAD = 8

_NDL = 2
_NH = _NUM_LEVELS - _NDL
_DS = tuple(_RES[lv] + 2 for lv in range(_NDL))
_DWORDS = tuple(2 * s * s * s + 16 for s in _DS)


def _pad_body(c0h, c1h, padh, c0b, c1b, rowb):
    nrows = padh.shape[0]
    per_w = nrows // _NW
    kc = c0b.shape[0]
    wid = lax.axis_index("s") * _NC + lax.axis_index("c")
    iota = lax.iota(jnp.int32, _L)
    zeros = iota * 0
    ones = zeros + 1

    @pl.loop(0, per_w // kc)
    def _chunk(k):
        base = wid * per_w + k * kc
        pltpu.sync_copy(c0h.at[pl.ds(base, kc)], c0b)
        pltpu.sync_copy(c1h.at[pl.ds(base, kc)], c1b)

        @pl.loop(0, kc // _L)
        def _g(g):
            rows = g * _L + iota
            plsc.store_scatter(rowb, [rows, zeros], c0b[pl.ds(g * _L, _L)])
            plsc.store_scatter(rowb, [rows, ones], c1b[pl.ds(g * _L, _L)])

        pltpu.sync_copy(rowb, padh.at[pl.ds(base, kc)])


def _pad_table(c0, c1):
    nrows = c0.shape[0]
    kc = 8192
    mesh = plsc.VectorSubcoreMesh(
        core_axis_name="c", subcore_axis_name="s", num_cores=_NC, num_subcores=_NS
    )
    f = pl.kernel(
        _pad_body,
        out_type=jax.ShapeDtypeStruct((nrows, _PAD), jnp.float32),
        mesh=mesh,
        scratch_types=[
            pltpu.VMEM((kc,), jnp.float32),
            pltpu.VMEM((kc,), jnp.float32),
            pltpu.VMEM((kc, _PAD), jnp.float32),
        ],
        compiler_params=pltpu.CompilerParams(
            needs_layout_passes=False, use_tc_tiling_on_sc=False
        ),
    )
    return f(c0, c1)


def _encode_body(
    x0h, x1h, x2h, tabh, ench,
    xb, wb, cidxb, idxb, featb, outb, d0b, d1b, sem0, sem1, xsem,
):
    n = x0h.shape[0]
    npts = n // _NW
    nchunks = npts // _C
    wid = lax.axis_index("s") * _NC + lax.axis_index("c")

    iota = lax.iota(jnp.int32, _L)
    dup = iota >> 1
    par = iota & 1
    sems = (sem0, sem1)

    def fire_x(k, p):
        base = wid * npts + k * _C
        o = p * 3 * _C
        c0 = pltpu.make_async_copy(
            x0h.at[pl.ds(base, _C)], xb.at[pl.ds(o, _C)], xsem)
        c1 = pltpu.make_async_copy(
            x1h.at[pl.ds(base, _C)], xb.at[pl.ds(o + _C, _C)], xsem)
        c2 = pltpu.make_async_copy(
            x2h.at[pl.ds(base, _C)], xb.at[pl.ds(o + 2 * _C, _C)], xsem)
        c0.start(); c1.start(); c2.start()

    def wait_x(k, p):
        base = wid * npts + k * _C
        o = p * 3 * _C
        pltpu.make_async_copy(
            x0h.at[pl.ds(base, _C)], xb.at[pl.ds(o, _C)], xsem).wait()
        pltpu.make_async_copy(
            x1h.at[pl.ds(base, _C)], xb.at[pl.ds(o + _C, _C)], xsem).wait()
        pltpu.make_async_copy(
            x2h.at[pl.ds(base, _C)], xb.at[pl.ds(o + 2 * _C, _C)], xsem).wait()

    def gather_copy(p):
        return pltpu.make_async_copy(
            tabh.at[idxb.at[pl.ds(p * 8 * _NH * _C, 8 * _NH * _C)]],
            featb.at[pl.ds(p * 8 * _NH * _C, 8 * _NH * _C)],
            sems[p],
        )

    def build_dense(dref, s, lv):
        s2 = s * s
        ng = (2 * s2 + _L - 1) // _L
        lvoff = lv << _LOG2_T

        @pl.loop(0, s)
        def _z(z):
            @pl.loop(0, s)
            def _y(y):
                hs = (y * _P1) ^ (z * _P2)
                pos0 = y * s
                h1 = ((iota ^ hs) & _MASK) | lvoff
                plsc.store_scatter(idxb, [pos0 + iota], h1)
                h2 = (((iota + _L) ^ hs) & _MASK) | lvoff
                plsc.store_scatter(
                    idxb, [pos0 + _L + iota], h2, mask=iota < (s - _L)
                )

            cp = pltpu.make_async_copy(
                tabh.at[idxb.at[pl.ds(0, s2)]], featb.at[pl.ds(0, s2)], sem0
            )
            cp.start()
            cp.wait()

            @pl.loop(0, ng)
            def _g(g):
                v = plsc.load_gather(featb, [g * 8 + dup, par])
                dref[pl.ds(z * 2 * s2 + g * _L, _L)] = v

    def pass1(p):
        xo = p * 3 * _C
        io = p * 8 * _NH * _C
        wo = p * 24 * _C
        co = p * _NDL * _C

        @pl.loop(0, _C // _L)
        def _p1(g):
            off = g * _L
            xr0 = (xb[pl.ds(xo + off, _L)] + 1.0) * 0.5
            xr1 = (xb[pl.ds(xo + _C + off, _L)] + 1.0) * 0.5
            xr2 = (xb[pl.ds(xo + 2 * _C + off, _L)] + 1.0) * 0.5
            for lv in range(_NUM_LEVELS):
                r = float(_RES[lv])
                iv = []
                for d, xr in enumerate((xr0, xr1, xr2)):
                    pos = xr * r
                    i_d = pos.astype(jnp.int32)
                    t = pos - i_d.astype(jnp.float32)
                    w = t * t * (3.0 - 2.0 * t)
                    wb[pl.ds(wo + lv * 3 * _C + d * _C + off, _L)] = w
                    iv.append(i_d)
                if lv < _NDL:
                    s = _DS[lv]
                    base2 = 2 * (iv[0] + s * iv[1] + (s * s) * iv[2])
                    cidxb[pl.ds(co + lv * _C + off, _L)] = base2
                else:
                    b0 = iv[1] * _P1
                    b1 = b0 + _P1
                    c0 = iv[2] * _P2
                    c1 = c0 + _P2
                    ab = (iv[0] ^ b0, (iv[0] + 1) ^ b0, iv[0] ^ b1,
                          (iv[0] + 1) ^ b1)
                    lvoff = lv << _LOG2_T
                    j = lv - _NDL
                    corner = 0
                    for cz in (c0, c1):
                        for abx in ab:
                            h = ((abx ^ cz) & _MASK) | lvoff
                            idxb[pl.ds(io + (j * 8 + corner) * _C + off, _L)] = h
                            corner += 1

    dense_refs = (d0b, d1b)

    def pass2(k, p):
        wo = p * 24 * _C
        fo = p * 8 * _NH * _C
        co = p * _NDL * _C
        cbase = (k % (128 // _C)) * _C

        @pl.loop(0, _C // 8)
        def _p2(j):
            p0 = j * 8
            rowv = p0 + dup
            wv = wo + p0 + dup
            for lv in range(_NUM_LEVELS):
                w = [
                    plsc.load_gather(wb, [wv + (lv * 3 * _C + d * _C)])
                    for d in range(3)
                ]
                if lv < _NDL:
                    s = _DS[lv]
                    bd = plsc.load_gather(cidxb, [co + lv * _C + p0 + dup])
                    bb = bd + par
                    f = [
                        plsc.load_gather(
                            dense_refs[lv],
                            [bb + 2 * (dx + s * dy + s * s * dz)],
                        )
                        for dz in (0, 1)
                        for dy in (0, 1)
                        for dx in (0, 1)
                    ]
                else:
                    jh = lv - _NDL
                    f = [
                        plsc.load_gather(
                            featb, [(fo + (jh * 8 + c) * _C + p0) + dup, par]
                        )
                        for c in range(8)
                    ]
                g00 = f[0] + w[0] * (f[1] - f[0])
                g10 = f[2] + w[0] * (f[3] - f[2])
                g01 = f[4] + w[0] * (f[5] - f[4])
                g11 = f[6] + w[0] * (f[7] - f[6])
                h0 = g00 + w[1] * (g10 - g00)
                h1 = g01 + w[1] * (g11 - g01)
                acc = h0 + w[2] * (h1 - h0)
                plsc.store_scatter(outb, [(2 * lv) + par, cbase + rowv], acc)

    def flush_out(k):
        base = wid * npts + (k - (128 // _C - 1)) * _C
        j0 = base // 128
        pltpu.sync_copy(outb.at[pl.ds(0, 8), :], ench.at[0, j0])
        pltpu.sync_copy(outb.at[pl.ds(8, 8), :], ench.at[1, j0])

    build_dense(d0b, _DS[0], 0)
    build_dense(d1b, _DS[1], 1)

    fire_x(0, 0)
    wait_x(0, 0)
    pass1(0)
    gather_copy(0).start()
    fire_x(1, 1)

    @pl.loop(0, nchunks // 2)
    def _chunk2(kk):
        ka = 2 * kk
        kb = 2 * kk + 1
        wait_x(kb, 1)
        pass1(1)
        gather_copy(1).start()

        @pl.when(kk < nchunks // 2 - 1)
        def _():
            fire_x(ka + 2, 0)

        gather_copy(0).wait()
        pass2(ka, 0)

        @pl.when(kk < nchunks // 2 - 1)
        def _():
            wait_x(ka + 2, 0)
            pass1(0)
            gather_copy(0).start()
            fire_x(kb + 2, 1)

        gather_copy(1).wait()
        pass2(kb, 1)

        @pl.when(kb % 4 == 3)
        def _():
            flush_out(kb)


def _encode(x0, x1, x2, table2):
    n = x0.shape[0]
    mesh = plsc.VectorSubcoreMesh(
        core_axis_name="c", subcore_axis_name="s", num_cores=_NC, num_subcores=_NS
    )
    f = pl.kernel(
        _encode_body,
        out_type=jax.ShapeDtypeStruct((2, n // 128, 8, 128), jnp.float32),
        mesh=mesh,
        scratch_types=[
            pltpu.VMEM((2 * 3 * _C,), jnp.float32),
            pltpu.VMEM((2 * 24 * _C,), jnp.float32),
            pltpu.VMEM((2 * _NDL * _C,), jnp.int32),
            pltpu.VMEM((2 * 8 * _NH * _C,), jnp.int32),
            pltpu.VMEM((2 * 8 * _NH * _C, _PAD), jnp.float32),
            pltpu.VMEM((2 * _NUM_LEVELS, 128), jnp.float32),
            pltpu.VMEM((_DWORDS[0],), jnp.float32),
            pltpu.VMEM((_DWORDS[1],), jnp.float32),
            pltpu.SemaphoreType.DMA,
            pltpu.SemaphoreType.DMA,
            pltpu.SemaphoreType.DMA,
        ],
        compiler_params=pltpu.CompilerParams(
            needs_layout_passes=False, use_tc_tiling_on_sc=False
        ),
    )
    return f(x0, x1, x2, table2)


def _mlp_body(xt_ref, enc_ref, w1xt_ref, w1gt_ref, b1_ref, w2t_ref, b2_ref, out_ref):
    nblk = enc_ref.shape[1]
    enc2d = jnp.concatenate(
        [
            jnp.concatenate([enc_ref[0, j], enc_ref[1, j]], axis=0)
            for j in range(nblk)
        ],
        axis=1,
    )
    h = (
        jnp.dot(w1xt_ref[...], xt_ref[...], preferred_element_type=jnp.float32)
        + jnp.dot(w1gt_ref[...], enc2d, preferred_element_type=jnp.float32)
        + b1_ref[...]
    )
    h = jnp.maximum(h, 0.0)
    o = jnp.dot(w2t_ref[...], h, preferred_element_type=jnp.float32) + b2_ref[...]
    out_ref[...] = jnp.maximum(o, 0.0) + jnp.log1p(jnp.exp(-jnp.abs(o)))


def _mlp(xt, enc, W1, b1, W2, b2):
    n = xt.shape[1]
    bn = 8192
    d_h = W1.shape[1]
    w1t = W1.T
    out = pl.pallas_call(
        _mlp_body,
        grid=(n // bn,),
        in_specs=[
            pl.BlockSpec((3, bn), lambda i: (0, i)),
            pl.BlockSpec((2, bn // 128, 8, 128), lambda i: (0, i, 0, 0)),
            pl.BlockSpec((d_h, 3), lambda i: (0, 0)),
            pl.BlockSpec((d_h, 2 * _NUM_LEVELS), lambda i: (0, 0)),
            pl.BlockSpec((d_h, 1), lambda i: (0, 0)),
            pl.BlockSpec((1, d_h), lambda i: (0, 0)),
            pl.BlockSpec((1, 1), lambda i: (0, 0)),
        ],
        out_specs=pl.BlockSpec((1, bn), lambda i: (0, i)),
        out_shape=jax.ShapeDtypeStruct((1, n), jnp.float32),
    )(xt, enc, w1t[:, :3], w1t[:, 3:], b1.reshape(d_h, 1), W2.T, b2.reshape(1, 1))
    return out.reshape(n, 1)


def kernel(x, table, W1, b1, W2, b2):
    xt = x.T
    tpair = table.reshape(_NUM_LEVELS * _T, _FD).T
    tablep = _pad_table(tpair[0], tpair[1])
    enc = _encode(xt[0], xt[1], xt[2], tablep)
    return _mlp(xt, enc, W1, b1, W2, b2)


# speedup vs baseline: 1.0561x; 1.0126x over previous
import functools

import jax
import jax.numpy as jnp
import numpy as np
from jax import lax
from jax.experimental import pallas as pl
from jax.experimental.pallas import tpu as pltpu
from jax.experimental.pallas import tpu_sc as plsc

_NUM_LEVELS = 8
_LOG2_T = 18
_T = 1 << _LOG2_T
_BASE_RES = 16
_MAX_RES = 1024
_SCALE = np.exp(np.log(_MAX_RES / _BASE_RES) / (_NUM_LEVELS - 1))
_RES = [int(np.floor(_BASE_RES * _SCALE ** lv)) for lv in range(_NUM_LEVELS)]
_P1 = int(np.int32(np.uint32(2654435761)))
_P2 = int(np.int32(805459861))
_MASK = _T - 1

_NC, _NS, _L = 2, 16, 16
_NW = _NC * _NS

_C = 32
_FD = 2
_PAD = 8

_NDL = 2
_NH = _NUM_LEVELS - _NDL
_DS = tuple(_RES[lv] + 2 for lv in range(_NDL))
_DWORDS = tuple(2 * s * s * s + 16 for s in _DS)


def _pad_body(c0h, c1h, padh, c0b, c1b, rowb):
    nrows = padh.shape[0]
    per_w = nrows // _NW
    kc = c0b.shape[0]
    wid = lax.axis_index("s") * _NC + lax.axis_index("c")
    iota = lax.iota(jnp.int32, _L)
    zeros = iota * 0
    ones = zeros + 1

    @pl.loop(0, per_w // kc)
    def _chunk(k):
        base = wid * per_w + k * kc
        pltpu.sync_copy(c0h.at[pl.ds(base, kc)], c0b)
        pltpu.sync_copy(c1h.at[pl.ds(base, kc)], c1b)

        @pl.loop(0, kc // _L)
        def _g(g):
            rows = g * _L + iota
            plsc.store_scatter(rowb, [rows, zeros], c0b[pl.ds(g * _L, _L)])
            plsc.store_scatter(rowb, [rows, ones], c1b[pl.ds(g * _L, _L)])

        pltpu.sync_copy(rowb, padh.at[pl.ds(base, kc)])


def _pad_table(c0, c1):
    nrows = c0.shape[0]
    kc = 8192
    mesh = plsc.VectorSubcoreMesh(
        core_axis_name="c", subcore_axis_name="s", num_cores=_NC, num_subcores=_NS
    )
    f = pl.kernel(
        _pad_body,
        out_type=jax.ShapeDtypeStruct((nrows, _PAD), jnp.float32),
        mesh=mesh,
        scratch_types=[
            pltpu.VMEM((kc,), jnp.float32),
            pltpu.VMEM((kc,), jnp.float32),
            pltpu.VMEM((kc, _PAD), jnp.float32),
        ],
        compiler_params=pltpu.CompilerParams(
            needs_layout_passes=False, use_tc_tiling_on_sc=False
        ),
    )
    return f(c0, c1)


def _encode_body(
    x0h, x1h, x2h, tabh, ench,
    xb, wb, cidxb, idxb, featb, outb, d0b, d1b, sem0, sem1, xsem,
):
    n = x0h.shape[0]
    npts = n // _NW
    nchunks = npts // _C
    wid = lax.axis_index("s") * _NC + lax.axis_index("c")

    iota = lax.iota(jnp.int32, _L)
    dup = iota >> 1
    par = iota & 1
    sems = (sem0, sem1)

    def fire_x(k, p):
        base = wid * npts + k * _C
        o = p * 3 * _C
        c0 = pltpu.make_async_copy(
            x0h.at[pl.ds(base, _C)], xb.at[pl.ds(o, _C)], xsem)
        c1 = pltpu.make_async_copy(
            x1h.at[pl.ds(base, _C)], xb.at[pl.ds(o + _C, _C)], xsem)
        c2 = pltpu.make_async_copy(
            x2h.at[pl.ds(base, _C)], xb.at[pl.ds(o + 2 * _C, _C)], xsem)
        c0.start(); c1.start(); c2.start()

    def wait_x(k, p):
        base = wid * npts + k * _C
        o = p * 3 * _C
        pltpu.make_async_copy(
            x0h.at[pl.ds(base, _C)], xb.at[pl.ds(o, _C)], xsem).wait()
        pltpu.make_async_copy(
            x1h.at[pl.ds(base, _C)], xb.at[pl.ds(o + _C, _C)], xsem).wait()
        pltpu.make_async_copy(
            x2h.at[pl.ds(base, _C)], xb.at[pl.ds(o + 2 * _C, _C)], xsem).wait()

    def gather_copy(p):
        return pltpu.make_async_copy(
            tabh.at[idxb.at[pl.ds(p * 8 * _NH * _C, 8 * _NH * _C)]],
            featb.at[pl.ds(p * 8 * _NH * _C, 8 * _NH * _C)],
            sems[p],
        )

    def build_dense(dref, s, lv):
        s2 = s * s
        ng = (2 * s2 + _L - 1) // _L
        lvoff = lv << _LOG2_T

        @pl.loop(0, s)
        def _z(z):
            @pl.loop(0, s)
            def _y(y):
                hs = (y * _P1) ^ (z * _P2)
                pos0 = y * s
                h1 = ((iota ^ hs) & _MASK) | lvoff
                plsc.store_scatter(idxb, [pos0 + iota], h1)
                h2 = (((iota + _L) ^ hs) & _MASK) | lvoff
                plsc.store_scatter(
                    idxb, [pos0 + _L + iota], h2, mask=iota < (s - _L)
                )

            cp = pltpu.make_async_copy(
                tabh.at[idxb.at[pl.ds(0, s2)]], featb.at[pl.ds(0, s2)], sem0
            )
            cp.start()
            cp.wait()

            @pl.loop(0, ng)
            def _g(g):
                v = plsc.load_gather(featb, [g * 8 + dup, par])
                dref[pl.ds(z * 2 * s2 + g * _L, _L)] = v

    def pass1(p):
        xo = p * 3 * _C
        io = p * 8 * _NH * _C
        wo = p * 24 * _C
        co = p * _NDL * _C

        @pl.loop(0, _C // _L)
        def _p1(g):
            off = g * _L
            xr0 = (xb[pl.ds(xo + off, _L)] + 1.0) * 0.5
            xr1 = (xb[pl.ds(xo + _C + off, _L)] + 1.0) * 0.5
            xr2 = (xb[pl.ds(xo + 2 * _C + off, _L)] + 1.0) * 0.5
            for lv in range(_NUM_LEVELS):
                r = float(_RES[lv])
                iv = []
                for d, xr in enumerate((xr0, xr1, xr2)):
                    pos = xr * r
                    i_d = pos.astype(jnp.int32)
                    t = pos - i_d.astype(jnp.float32)
                    w = t * t * (3.0 - 2.0 * t)
                    wb[pl.ds(wo + lv * 3 * _C + d * _C + off, _L)] = w
                    iv.append(i_d)
                if lv < _NDL:
                    s = _DS[lv]
                    base2 = 2 * (iv[0] + s * iv[1] + (s * s) * iv[2])
                    cidxb[pl.ds(co + lv * _C + off, _L)] = base2
                else:
                    b0 = iv[1] * _P1
                    b1 = b0 + _P1
                    c0 = iv[2] * _P2
                    c1 = c0 + _P2
                    ab = (iv[0] ^ b0, (iv[0] + 1) ^ b0, iv[0] ^ b1,
                          (iv[0] + 1) ^ b1)
                    lvoff = lv << _LOG2_T
                    j = lv - _NDL
                    corner = 0
                    for cz in (c0, c1):
                        for abx in ab:
                            h = ((abx ^ cz) & _MASK) | lvoff
                            idxb[pl.ds(io + (j * 8 + corner) * _C + off, _L)] = h
                            corner += 1

    dense_refs = (d0b, d1b)

    def pass2(k, p):
        wo = p * 24 * _C
        fo = p * 8 * _NH * _C
        co = p * _NDL * _C
        cbase = (k % (128 // _C)) * _C

        @pl.loop(0, _C // 8)
        def _p2(j):
            p0 = j * 8
            rowv = p0 + dup
            wv = wo + p0 + dup
            for lv in range(_NUM_LEVELS):
                w = [
                    plsc.load_gather(wb, [wv + (lv * 3 * _C + d * _C)])
                    for d in range(3)
                ]
                if lv < _NDL:
                    s = _DS[lv]
                    bd = plsc.load_gather(cidxb, [co + lv * _C + p0 + dup])
                    bb = bd + par
                    f = [
                        plsc.load_gather(
                            dense_refs[lv],
                            [bb + 2 * (dx + s * dy + s * s * dz)],
                        )
                        for dz in (0, 1)
                        for dy in (0, 1)
                        for dx in (0, 1)
                    ]
                else:
                    jh = lv - _NDL
                    f = [
                        plsc.load_gather(
                            featb, [(fo + (jh * 8 + c) * _C + p0) + dup, par]
                        )
                        for c in range(8)
                    ]
                g00 = f[0] + w[0] * (f[1] - f[0])
                g10 = f[2] + w[0] * (f[3] - f[2])
                g01 = f[4] + w[0] * (f[5] - f[4])
                g11 = f[6] + w[0] * (f[7] - f[6])
                h0 = g00 + w[1] * (g10 - g00)
                h1 = g01 + w[1] * (g11 - g01)
                acc = h0 + w[2] * (h1 - h0)
                plsc.store_scatter(outb, [(2 * lv) + par, cbase + rowv], acc)

    def flush_out(k):
        base = wid * npts + (k - (128 // _C - 1)) * _C
        j0 = base // 128
        pltpu.sync_copy(outb.at[pl.ds(0, 8), :], ench.at[0, j0])
        pltpu.sync_copy(outb.at[pl.ds(8, 8), :], ench.at[1, j0])

    build_dense(d0b, _DS[0], 0)
    build_dense(d1b, _DS[1], 1)

    fire_x(0, 0)
    wait_x(0, 0)
    pass1(0)
    gather_copy(0).start()
    fire_x(1, 1)

    @pl.loop(0, nchunks // 2)
    def _chunk2(kk):
        ka = 2 * kk
        kb = 2 * kk + 1
        wait_x(kb, 1)
        pass1(1)
        gather_copy(1).start()

        @pl.when(kk < nchunks // 2 - 1)
        def _():
            fire_x(ka + 2, 0)

        gather_copy(0).wait()
        pass2(ka, 0)

        @pl.when(kk < nchunks // 2 - 1)
        def _():
            wait_x(ka + 2, 0)
            pass1(0)
            gather_copy(0).start()
            fire_x(kb + 2, 1)

        gather_copy(1).wait()
        pass2(kb, 1)

        @pl.when(kb % 4 == 3)
        def _():
            flush_out(kb)


def _encode(x0, x1, x2, table2):
    n = x0.shape[0]
    mesh = plsc.VectorSubcoreMesh(
        core_axis_name="c", subcore_axis_name="s", num_cores=_NC, num_subcores=_NS
    )
    f = pl.kernel(
        _encode_body,
        out_type=jax.ShapeDtypeStruct((2, n // 128, 8, 128), jnp.float32),
        mesh=mesh,
        scratch_types=[
            pltpu.VMEM((2 * 3 * _C,), jnp.float32),
            pltpu.VMEM((2 * 24 * _C,), jnp.float32),
            pltpu.VMEM((2 * _NDL * _C,), jnp.int32),
            pltpu.VMEM((2 * 8 * _NH * _C,), jnp.int32),
            pltpu.VMEM((2 * 8 * _NH * _C, _PAD), jnp.float32),
            pltpu.VMEM((2 * _NUM_LEVELS, 128), jnp.float32),
            pltpu.VMEM((_DWORDS[0],), jnp.float32),
            pltpu.VMEM((_DWORDS[1],), jnp.float32),
            pltpu.SemaphoreType.DMA,
            pltpu.SemaphoreType.DMA,
            pltpu.SemaphoreType.DMA,
        ],
        compiler_params=pltpu.CompilerParams(
            needs_layout_passes=False, use_tc_tiling_on_sc=False
        ),
    )
    return f(x0, x1, x2, table2)


def _mlp_body(xt_ref, enc_ref, w1xt_ref, w1gt_ref, b1_ref, w2t_ref, b2_ref, out_ref):
    nblk = enc_ref.shape[1]
    enc2d = jnp.concatenate(
        [
            jnp.concatenate([enc_ref[0, j], enc_ref[1, j]], axis=0)
            for j in range(nblk)
        ],
        axis=1,
    )
    h = (
        jnp.dot(w1xt_ref[...], xt_ref[...], preferred_element_type=jnp.float32)
        + jnp.dot(w1gt_ref[...], enc2d, preferred_element_type=jnp.float32)
        + b1_ref[...]
    )
    h = jnp.maximum(h, 0.0)
    o = jnp.dot(w2t_ref[...], h, preferred_element_type=jnp.float32) + b2_ref[...]
    out_ref[...] = jnp.maximum(o, 0.0) + jnp.log1p(jnp.exp(-jnp.abs(o)))


def _mlp(xt, enc, W1, b1, W2, b2):
    n = xt.shape[1]
    bn = 16384
    d_h = W1.shape[1]
    w1t = W1.T
    out = pl.pallas_call(
        _mlp_body,
        grid=(n // bn,),
        in_specs=[
            pl.BlockSpec((3, bn), lambda i: (0, i)),
            pl.BlockSpec((2, bn // 128, 8, 128), lambda i: (0, i, 0, 0)),
            pl.BlockSpec((d_h, 3), lambda i: (0, 0)),
            pl.BlockSpec((d_h, 2 * _NUM_LEVELS), lambda i: (0, 0)),
            pl.BlockSpec((d_h, 1), lambda i: (0, 0)),
            pl.BlockSpec((1, d_h), lambda i: (0, 0)),
            pl.BlockSpec((1, 1), lambda i: (0, 0)),
        ],
        out_specs=pl.BlockSpec((1, bn), lambda i: (0, i)),
        out_shape=jax.ShapeDtypeStruct((1, n), jnp.float32),
    )(xt, enc, w1t[:, :3], w1t[:, 3:], b1.reshape(d_h, 1), W2.T, b2.reshape(1, 1))
    return out.reshape(n, 1)


def kernel(x, table, W1, b1, W2, b2):
    xt = x.T
    tpair = table.reshape(_NUM_LEVELS * _T, _FD).T
    tablep = _pad_table(tpair[0], tpair[1])
    enc = _encode(xt[0], xt[1], xt[2], tablep)
    return _mlp(xt, enc, W1, b1, W2, b2)
